# Initial kernel scaffold; baseline (speedup 1.0000x reference)
import jax
import jax.numpy as jnp
from jax.experimental import pallas as pl


def kernel(frames, target, W_pred, b_pred, emb_table, bins):
    raise NotImplementedError("write your pallas kernel here")


# speedup vs baseline: 11.0847x; 11.0847x over previous
import functools

import jax
import jax.numpy as jnp
from jax import lax
from jax.experimental import pallas as pl
from jax.experimental.pallas import tpu as pltpu
from jax.experimental.pallas import tpu_sc as plsc

B, T, ENC_DIM = 16, 4096, 512
N_BINS = 256
NTOT = B * T
NC, NS, L = 2, 16, 16
NW = NC * NS
PER_W = NTOT // NW
CHUNK = 64
NCHUNK = PER_W // CHUNK


def _sc_body(cls_hbm, table_hbm, out_hbm, cls_v, rows0, rows1, gsem):
    wid = lax.axis_index("s") * NC + lax.axis_index("c")
    base = wid * PER_W

    pltpu.sync_copy(cls_hbm.at[pl.ds(base, PER_W)], cls_v.at[pl.ds(0, PER_W)])
    for j in range(CHUNK // L):
        cls_v[pl.ds(PER_W + j * L, L)] = jnp.zeros((L,), jnp.int32)

    def gather_src(g):
        return table_hbm.at[cls_v.at[pl.ds(g * CHUNK, CHUNK)]]

    pltpu.async_copy(gather_src(0), rows0, gsem)

    def chunk_body(g2, _):
        for b2, (buf, nbuf) in enumerate(((rows0, rows1), (rows1, rows0))):
            g = g2 * 2 + b2
            pltpu.make_async_copy(gather_src(g), buf, gsem).wait()
            pltpu.async_copy(gather_src(g + 1), nbuf, gsem)
            pltpu.sync_copy(buf, out_hbm.at[pl.ds(base + g * CHUNK, CHUNK)])
        return 0

    lax.fori_loop(0, NCHUNK // 2, chunk_body, 0)
    pltpu.make_async_copy(gather_src(NCHUNK), rows0, gsem).wait()


_sc_lookup = functools.partial(
    pl.kernel,
    out_type=jax.ShapeDtypeStruct((NTOT, ENC_DIM), jnp.float32),
    mesh=plsc.VectorSubcoreMesh(core_axis_name="c", subcore_axis_name="s",
                                num_cores=NC, num_subcores=NS),
    scratch_types=[
        pltpu.VMEM((PER_W + CHUNK,), jnp.int32),
        pltpu.VMEM((CHUNK, ENC_DIM), jnp.float32),
        pltpu.VMEM((CHUNK, ENC_DIM), jnp.float32),
        pltpu.SemaphoreType.DMA,
    ],
)(_sc_body)


_CLS_BT = 8192


def _tc_cls_body(t_ref, bins_ref, ones_ref, c_ref):
    maskf = (bins_ref[...] < t_ref[...]).astype(jnp.float32)
    c_ref[...] = jnp.dot(maskf, ones_ref[...],
                         preferred_element_type=jnp.float32).astype(jnp.int32)


def _tc_classes(t2d, bins_row, ones8):
    return pl.pallas_call(
        _tc_cls_body,
        grid=(NTOT // _CLS_BT,),
        in_specs=[
            pl.BlockSpec((_CLS_BT, 1), lambda i: (i, 0)),
            pl.BlockSpec((1, N_BINS), lambda i: (0, 0)),
            pl.BlockSpec((N_BINS, 8), lambda i: (0, 0)),
        ],
        out_specs=pl.BlockSpec((_CLS_BT, 8), lambda i: (i, 0)),
        out_shape=jax.ShapeDtypeStruct((NTOT, 8), jnp.int32),
    )(t2d, bins_row, ones8)


_TC_BT = 4096


def _tc_pred_body(f_ref, w_ref, b_ref, o_ref):
    o_ref[...] = jnp.dot(f_ref[...], w_ref[...],
                         preferred_element_type=jnp.float32) + b_ref[0, 0]


def _tc_pred(frames2d, w8, b2d):
    return pl.pallas_call(
        _tc_pred_body,
        grid=(NTOT // _TC_BT,),
        in_specs=[
            pl.BlockSpec((_TC_BT, ENC_DIM), lambda i: (i, 0)),
            pl.BlockSpec((ENC_DIM, 8), lambda i: (0, 0)),
            pl.BlockSpec((1, 1), lambda i: (0, 0)),
        ],
        out_specs=pl.BlockSpec((_TC_BT, 8), lambda i: (i, 0)),
        out_shape=jax.ShapeDtypeStruct((NTOT, 8), jnp.float32),
    )(frames2d, w8, b2d)


def kernel(frames, target, W_pred, b_pred, emb_table, bins):
    bins_row = jnp.concatenate(
        [bins, jnp.full((1,), jnp.inf, jnp.float32)]).reshape(1, N_BINS)
    ones8 = jnp.ones((N_BINS, 8), jnp.float32)
    classes8 = _tc_classes(target.reshape(NTOT, 1), bins_row, ones8)
    classes = classes8[:, 0]

    emb_flat = _sc_lookup(classes, emb_table)
    emb = emb_flat.reshape(B, T, ENC_DIM)

    frames2d = frames.reshape(NTOT, ENC_DIM)
    w8 = jnp.concatenate(
        [W_pred, jnp.zeros((ENC_DIM, 7), jnp.float32)], axis=1)
    pred8 = _tc_pred(frames2d, w8, b_pred.reshape(1, 1))
    prediction = pred8[:, 0].reshape(B, T)
    return (prediction, emb)
